# baseline (speedup 1.0000x reference)
import functools

import jax
import jax.numpy as jnp
from jax import lax
from jax.experimental import pallas as pl
from jax.experimental.pallas import tpu as pltpu
from jax.experimental.pallas import tpu_sc as plsc

EMB = 64
PAD = 128
LANES = 128
NBUF = 2


def _make_emb_kernel(b_total, l_seq, nc, ns):
    nw = nc * ns
    rows_per_w = b_total // nw
    c0 = min(LANES, l_seq)
    c1 = l_seq - c0
    mesh = plsc.VectorSubcoreMesh(core_axis_name="c", subcore_axis_name="s")

    @functools.partial(
        pl.kernel,
        mesh=mesh,
        out_type=jax.ShapeDtypeStruct((b_total, l_seq, PAD), jnp.float32),
        compiler_params=pltpu.CompilerParams(use_tc_tiling_on_sc=False),
        scratch_types=(
            [pltpu.VMEM((rows_per_w, LANES), jnp.int32)] * 2
            + [pltpu.VMEM((l_seq, PAD), jnp.float32)] * NBUF
            + [pltpu.SemaphoreType.DMA] * (2 * NBUF)
        ),
    )
    def emb_kernel(xa_hbm, xb_hbm, table_hbm, out_hbm, idxa_v, idxb_v, *bufs_and_sems):
        rows = bufs_and_sems[:NBUF]
        gsem = bufs_and_sems[NBUF : 2 * NBUF]
        ssem = bufs_and_sems[2 * NBUF :]

        wid = lax.axis_index("s") * nc + lax.axis_index("c")
        base = wid * rows_per_w

        pltpu.sync_copy(xa_hbm.at[pl.ds(base, rows_per_w)], idxa_v)
        pltpu.sync_copy(xb_hbm.at[pl.ds(base, rows_per_w)], idxb_v)

        def gather_start(bi, buf):
            pltpu.make_async_copy(
                table_hbm.at[idxa_v.at[bi]],
                rows[buf].at[pl.ds(0, c0)],
                gsem[buf],
            ).start()
            if c1:
                pltpu.make_async_copy(
                    table_hbm.at[idxb_v.at[bi, pl.ds(0, c1)]],
                    rows[buf].at[pl.ds(c0, c1)],
                    gsem[buf],
                ).start()

        def gather_wait(buf):
            pltpu.make_async_copy(
                table_hbm.at[idxa_v.at[0, pl.ds(0, l_seq)]], rows[buf], gsem[buf]
            ).wait()

        def store_start(bi, buf):
            pltpu.make_async_copy(
                rows[buf], out_hbm.at[base + bi], ssem[buf]
            ).start()

        def store_wait(buf):
            pltpu.make_async_copy(
                rows[buf], out_hbm.at[base], ssem[buf]
            ).wait()

        for b in range(NBUF):
            gather_start(b, b)

        n_main = rows_per_w // NBUF - 1

        def body(i, _):
            for b in range(NBUF):
                bi = i * NBUF + b
                gather_wait(b)
                store_start(bi, b)
                store_wait(b)
                gather_start(bi + NBUF, b)
            return 0

        lax.fori_loop(0, n_main, body, 0)

        for b in range(NBUF):
            bi = n_main * NBUF + b
            gather_wait(b)
            store_start(bi, b)
            store_wait(b)

    return emb_kernel


def kernel(x, table):
    b, l = x.shape
    info = plsc.get_sparse_core_info()
    nc, ns = info.num_cores, info.num_subcores
    xi = x.astype(jnp.int32)
    xa = lax.slice(xi, (0, 0), (b, LANES))
    xb = lax.pad(
        lax.slice(xi, (0, LANES), (b, l)),
        jnp.int32(0),
        ((0, 0, 0), (0, 2 * LANES - l, 0)),
    )
    tpad = lax.pad(table, jnp.float32(0), ((0, 0, 0), (0, PAD - EMB, 0)))
    emb = _make_emb_kernel(b, l, nc, ns)
    out = emb(xa, xb, tpad)
    return out[:, :, :EMB]


# speedup vs baseline: 1.1763x; 1.1763x over previous
import functools

import jax
import jax.numpy as jnp
from jax import lax
from jax.experimental import pallas as pl
from jax.experimental.pallas import tpu as pltpu
from jax.experimental.pallas import tpu_sc as plsc

EMB = 64
PAD = 128
LANES = 128
NBUF = 4


def _make_emb_kernel(b_total, l_seq, nc, ns):
    nw = nc * ns
    rows_per_w = b_total // nw
    c0 = min(LANES, l_seq)
    c1 = l_seq - c0
    mesh = plsc.VectorSubcoreMesh(core_axis_name="c", subcore_axis_name="s")

    @functools.partial(
        pl.kernel,
        mesh=mesh,
        out_type=jax.ShapeDtypeStruct((b_total, l_seq, PAD), jnp.float32),
        compiler_params=pltpu.CompilerParams(use_tc_tiling_on_sc=False),
        scratch_types=(
            [pltpu.VMEM((rows_per_w, LANES), jnp.int32)] * 2
            + [pltpu.VMEM((l_seq, EMB), jnp.float32)] * NBUF
            + [pltpu.SemaphoreType.DMA] * (2 * NBUF)
        ),
    )
    def emb_kernel(xa_hbm, xb_hbm, table_hbm, out_hbm, idxa_v, idxb_v, *bufs_and_sems):
        rows = bufs_and_sems[:NBUF]
        gsem = bufs_and_sems[NBUF : 2 * NBUF]
        ssem = bufs_and_sems[2 * NBUF :]

        wid = lax.axis_index("s") * nc + lax.axis_index("c")
        base = wid * rows_per_w

        pltpu.sync_copy(xa_hbm.at[pl.ds(base, rows_per_w)], idxa_v)
        pltpu.sync_copy(xb_hbm.at[pl.ds(base, rows_per_w)], idxb_v)

        def gather_start(bi, buf):
            pltpu.make_async_copy(
                table_hbm.at[idxa_v.at[bi]],
                rows[buf].at[pl.ds(0, c0)],
                gsem[buf],
            ).start()
            if c1:
                pltpu.make_async_copy(
                    table_hbm.at[idxb_v.at[bi, pl.ds(0, c1)]],
                    rows[buf].at[pl.ds(c0, c1)],
                    gsem[buf],
                ).start()

        def gather_wait(buf):
            pltpu.make_async_copy(
                table_hbm.at[idxa_v.at[0, pl.ds(0, l_seq)]], rows[buf], gsem[buf]
            ).wait()

        def store_start(bi, buf):
            pltpu.make_async_copy(
                rows[buf], out_hbm.at[base + bi, :, pl.ds(0, EMB)], ssem[buf]
            ).start()

        def store_wait(buf):
            pltpu.make_async_copy(
                rows[buf], out_hbm.at[base, :, pl.ds(0, EMB)], ssem[buf]
            ).wait()

        for b in range(NBUF):
            gather_start(b, b)

        n_main = rows_per_w // NBUF - 1

        def body(i, _):
            for b in range(NBUF):
                bi = i * NBUF + b
                gather_wait(b)
                store_start(bi, b)
                store_wait(b)
                gather_start(bi + NBUF, b)
            return 0

        lax.fori_loop(0, n_main, body, 0)

        for b in range(NBUF):
            bi = n_main * NBUF + b
            gather_wait(b)
            store_start(bi, b)
            store_wait(b)

    return emb_kernel


def kernel(x, table):
    b, l = x.shape
    info = plsc.get_sparse_core_info()
    nc, ns = info.num_cores, info.num_subcores
    xi = x.astype(jnp.int32)
    xa = lax.slice(xi, (0, 0), (b, LANES))
    xb = lax.pad(
        lax.slice(xi, (0, LANES), (b, l)),
        jnp.int32(0),
        ((0, 0, 0), (0, 2 * LANES - l, 0)),
    )
    v = table.shape[0]
    t2 = lax.pad(table, jnp.float32(0), ((0, 0, 0), (0, PAD - EMB, 0))).reshape(
        2 * v, EMB
    )
    emb = _make_emb_kernel(b, l, nc, ns)
    out = emb(xa * 2, xb * 2, t2)
    return out[:, :, :EMB]
